# Initial kernel scaffold; baseline (speedup 1.0000x reference)
import functools

import jax
import jax.numpy as jnp
from jax import lax
from jax.experimental import pallas as pl
from jax.experimental.pallas import tpu as pltpu
from jax.experimental.pallas import tpu_sc as plsc

N = 10000
E = 320000
NC, NS = 2, 16
NW = NC * NS
CH = 128
NCHUNK = 79
E_PAD = NW * NCHUNK * CH
AGG_ROWS = N + 16
ZR = AGG_ROWS // NS
DR = N // NS
BR = 1000


def _spmm_sc(D):
    mesh = plsc.VectorSubcoreMesh(core_axis_name="c", subcore_axis_name="s")

    @functools.partial(
        pl.kernel,
        out_type=jax.ShapeDtypeStruct((NC, N, D), jnp.float32),
        mesh=mesh,
        scratch_types=[
            pltpu.VMEM((NCHUNK, CH), jnp.int32),
            pltpu.VMEM((NCHUNK, CH), jnp.int32),
            pltpu.VMEM((CH, D), jnp.float32),
            pltpu.VMEM_SHARED((AGG_ROWS, D), jnp.float32),
            pltpu.SemaphoreType.DMA,
        ],
    )
    def spmm(y_hbm, srcp_hbm, dstp_hbm, zeros_hbm, out_hbm,
             src_v, dst_v, buf, agg, sem):
        c = lax.axis_index("c")
        s = lax.axis_index("s")
        wid = s * NC + c
        pltpu.sync_copy(zeros_hbm.at[pl.ds(s * ZR, ZR)], agg.at[pl.ds(s * ZR, ZR)])
        pltpu.sync_copy(srcp_hbm.at[wid], src_v)
        pltpu.sync_copy(dstp_hbm.at[wid], dst_v)
        plsc.subcore_barrier()

        def body(j, carry):
            pltpu.async_copy(y_hbm.at[src_v.at[j]], buf, sem).wait()
            pltpu.sync_copy(buf, agg.at[dst_v.at[j]], add=True)
            return carry

        lax.fori_loop(0, NCHUNK, body, 0)
        plsc.subcore_barrier()
        pltpu.sync_copy(agg.at[pl.ds(s * DR, DR)], out_hbm.at[c, pl.ds(s * DR, DR)])

    return spmm


def _deg_sc():
    mesh = plsc.VectorSubcoreMesh(core_axis_name="c", subcore_axis_name="s")
    D = 16

    @functools.partial(
        pl.kernel,
        out_type=jax.ShapeDtypeStruct((NC, N, D), jnp.float32),
        mesh=mesh,
        scratch_types=[
            pltpu.VMEM((NCHUNK, CH), jnp.int32),
            pltpu.VMEM((CH, D), jnp.float32),
            pltpu.VMEM_SHARED((AGG_ROWS, D), jnp.float32),
        ],
    )
    def degk(onescol_hbm, dstp_hbm, zeros_hbm, out_hbm, dst_v, buf, agg):
        c = lax.axis_index("c")
        s = lax.axis_index("s")
        wid = s * NC + c
        pltpu.sync_copy(zeros_hbm.at[pl.ds(s * ZR, ZR)], agg.at[pl.ds(s * ZR, ZR)])
        pltpu.sync_copy(onescol_hbm, buf)
        pltpu.sync_copy(dstp_hbm.at[wid], dst_v)
        plsc.subcore_barrier()

        def body(j, carry):
            pltpu.sync_copy(buf, agg.at[dst_v.at[j]], add=True)
            return carry

        lax.fori_loop(0, NCHUNK, body, 0)
        plsc.subcore_barrier()
        pltpu.sync_copy(agg.at[pl.ds(s * DR, DR)], out_hbm.at[c, pl.ds(s * DR, DR)])

    return degk


def _dot(a, b):
    return lax.dot_general(a, b, (((1,), (0,)), ((), ())),
                           precision=lax.Precision.HIGHEST,
                           preferred_element_type=jnp.float32)


def _tc_first(din, dout):
    def body(x_ref, w_ref, degs_ref, y_ref, dinv_ref):
        deg = degs_ref[0, :, 0:1] + degs_ref[1, :, 0:1] + 1.0
        dinv = lax.rsqrt(deg)
        y_ref[...] = _dot(x_ref[...], w_ref[...]) * dinv
        dinv_ref[...] = dinv

    return pl.pallas_call(
        body,
        grid=(N // BR,),
        in_specs=[
            pl.BlockSpec((BR, din), lambda i: (i, 0)),
            pl.BlockSpec((din, dout), lambda i: (0, 0)),
            pl.BlockSpec((2, BR, 16), lambda i: (0, i, 0)),
        ],
        out_specs=[
            pl.BlockSpec((BR, dout), lambda i: (i, 0)),
            pl.BlockSpec((BR, 1), lambda i: (i, 0)),
        ],
        out_shape=[
            jax.ShapeDtypeStruct((N, dout), jnp.float32),
            jax.ShapeDtypeStruct((N, 1), jnp.float32),
        ],
    )


def _tc_mid(din, dout):
    def body(raw_ref, y_ref, dinv_ref, b_ref, w_ref, out_ref):
        dinv = dinv_ref[...]
        acc = raw_ref[0] + raw_ref[1] + y_ref[...]
        X = jnp.maximum(acc * dinv + b_ref[...], 0.0)
        out_ref[...] = _dot(X, w_ref[...]) * dinv

    return pl.pallas_call(
        body,
        grid=(N // BR,),
        in_specs=[
            pl.BlockSpec((2, BR, din), lambda i: (0, i, 0)),
            pl.BlockSpec((BR, din), lambda i: (i, 0)),
            pl.BlockSpec((BR, 1), lambda i: (i, 0)),
            pl.BlockSpec((1, din), lambda i: (0, 0)),
            pl.BlockSpec((din, dout), lambda i: (0, 0)),
        ],
        out_specs=pl.BlockSpec((BR, dout), lambda i: (i, 0)),
        out_shape=jax.ShapeDtypeStruct((N, dout), jnp.float32),
    )


def _tc_last(din):
    def body(raw_ref, y_ref, dinv_ref, b_ref, out_ref):
        dinv = dinv_ref[...]
        acc = raw_ref[0] + raw_ref[1] + y_ref[...]
        X = jnp.maximum(acc * dinv + b_ref[...], 0.0)
        m = jnp.max(X, axis=1, keepdims=True)
        lse = jnp.log(jnp.sum(jnp.exp(X - m), axis=1, keepdims=True)) + m
        out_ref[...] = X - lse

    return pl.pallas_call(
        body,
        grid=(N // BR,),
        in_specs=[
            pl.BlockSpec((2, BR, din), lambda i: (0, i, 0)),
            pl.BlockSpec((BR, din), lambda i: (i, 0)),
            pl.BlockSpec((BR, 1), lambda i: (i, 0)),
            pl.BlockSpec((1, din), lambda i: (0, 0)),
        ],
        out_specs=pl.BlockSpec((BR, din), lambda i: (i, 0)),
        out_shape=jax.ShapeDtypeStruct((N, din), jnp.float32),
    )


def kernel(x, edge_index, W0, b0, W1, b1, W2, b2, W3, b3, W4, b4):
    src = edge_index[0]
    dst = edge_index[1]
    pad = E_PAD - E
    srcp = jnp.concatenate([src, jnp.zeros((pad,), jnp.int32)]).reshape(NW, NCHUNK, CH)
    dstp = jnp.concatenate([dst, jnp.full((pad,), N, jnp.int32)]).reshape(NW, NCHUNK, CH)
    onescol = jnp.concatenate(
        [jnp.ones((CH, 1), jnp.float32), jnp.zeros((CH, 15), jnp.float32)], axis=1)
    zeros = {d: jnp.zeros((AGG_ROWS, d), jnp.float32) for d in (128, 64, 32, 16)}

    degp = _deg_sc()(onescol, dstp, zeros[16])
    y, dinv = _tc_first(128, 128)(x, W0, degp)

    params = [(b0, W1, 64), (b1, W2, 32), (b2, W3, 16), (b3, W4, 16)]
    din = 128
    for b, W, dout in params:
        raw = _spmm_sc(din)(y, srcp, dstp, zeros[din])
        y = _tc_mid(din, dout)(raw, y, dinv, b.reshape(1, din), W)
        din = dout
    raw = _spmm_sc(din)(y, srcp, dstp, zeros[din])
    return _tc_last(din)(raw, y, dinv, b4.reshape(1, din))


# speedup vs baseline: 12.1665x; 12.1665x over previous
import functools

import jax
import jax.numpy as jnp
from jax import lax
from jax.experimental import pallas as pl
from jax.experimental.pallas import tpu as pltpu
from jax.experimental.pallas import tpu_sc as plsc

N = 10000
E = 320000
NC, NS = 2, 16
NW = NC * NS
CH = 128
NCHUNK = 79
E_PAD = NW * NCHUNK * CH
AGG_ROWS = 10112
ZR = AGG_ROWS // NS
BR = 1000


def _spmm_sc(D):
    mesh = plsc.VectorSubcoreMesh(core_axis_name="c", subcore_axis_name="s")

    @functools.partial(
        pl.kernel,
        out_type=jax.ShapeDtypeStruct((NC, AGG_ROWS, D), jnp.float32),
        mesh=mesh,
        scratch_types=[
            pltpu.VMEM((NCHUNK, CH), jnp.int32),
            pltpu.VMEM((NCHUNK, CH), jnp.int32),
            pltpu.VMEM((CH, D), jnp.float32),
            pltpu.VMEM_SHARED((AGG_ROWS, D), jnp.float32),
            pltpu.SemaphoreType.DMA,
        ],
        compiler_params=pltpu.CompilerParams(use_tc_tiling_on_sc=False),
    )
    def spmm(y_hbm, srcp_hbm, dstp_hbm, zeros_hbm, out_hbm,
             src_v, dst_v, buf, agg, sem):
        c = lax.axis_index("c")
        s = lax.axis_index("s")
        wid = s * NC + c
        pltpu.sync_copy(zeros_hbm.at[pl.ds(s * ZR, ZR)], agg.at[pl.ds(s * ZR, ZR)])
        pltpu.sync_copy(srcp_hbm.at[wid], src_v)
        pltpu.sync_copy(dstp_hbm.at[wid], dst_v)
        plsc.subcore_barrier()

        def body(j, carry):
            pltpu.async_copy(y_hbm.at[src_v.at[j]], buf, sem).wait()
            pltpu.sync_copy(buf, agg.at[dst_v.at[j]], add=True)
            return carry

        lax.fori_loop(0, NCHUNK, body, 0)
        plsc.subcore_barrier()
        pltpu.sync_copy(agg.at[pl.ds(s * ZR, ZR)], out_hbm.at[c, pl.ds(s * ZR, ZR)])

    return spmm


def _deg_sc():
    mesh = plsc.VectorSubcoreMesh(core_axis_name="c", subcore_axis_name="s")
    D = 16

    @functools.partial(
        pl.kernel,
        out_type=jax.ShapeDtypeStruct((NC, AGG_ROWS, D), jnp.float32),
        mesh=mesh,
        scratch_types=[
            pltpu.VMEM((NCHUNK, CH), jnp.int32),
            pltpu.VMEM((CH, D), jnp.float32),
            pltpu.VMEM_SHARED((AGG_ROWS, D), jnp.float32),
        ],
        compiler_params=pltpu.CompilerParams(use_tc_tiling_on_sc=False),
    )
    def degk(onescol_hbm, dstp_hbm, zeros_hbm, out_hbm, dst_v, buf, agg):
        c = lax.axis_index("c")
        s = lax.axis_index("s")
        wid = s * NC + c
        pltpu.sync_copy(zeros_hbm.at[pl.ds(s * ZR, ZR)], agg.at[pl.ds(s * ZR, ZR)])
        pltpu.sync_copy(onescol_hbm, buf)
        pltpu.sync_copy(dstp_hbm.at[wid], dst_v)
        plsc.subcore_barrier()

        def body(j, carry):
            pltpu.sync_copy(buf, agg.at[dst_v.at[j]], add=True)
            return carry

        lax.fori_loop(0, NCHUNK, body, 0)
        plsc.subcore_barrier()
        pltpu.sync_copy(agg.at[pl.ds(s * ZR, ZR)], out_hbm.at[c, pl.ds(s * ZR, ZR)])

    return degk


def _dot(a, b):
    return lax.dot_general(a, b, (((1,), (0,)), ((), ())),
                           precision=lax.Precision.HIGHEST,
                           preferred_element_type=jnp.float32)


def _tc_first(din, dout):
    def body(x_ref, w_ref, degs_ref, y_ref, dinv_ref):
        deg = degs_ref[0, :, 0:1] + degs_ref[1, :, 0:1] + 1.0
        dinv = lax.rsqrt(deg)
        y_ref[...] = _dot(x_ref[...], w_ref[...]) * dinv
        dinv_ref[...] = dinv

    return pl.pallas_call(
        body,
        grid=(N // BR,),
        in_specs=[
            pl.BlockSpec((BR, din), lambda i: (i, 0)),
            pl.BlockSpec((din, dout), lambda i: (0, 0)),
            pl.BlockSpec((2, BR, 16), lambda i: (0, i, 0)),
        ],
        out_specs=[
            pl.BlockSpec((BR, dout), lambda i: (i, 0)),
            pl.BlockSpec((BR, 1), lambda i: (i, 0)),
        ],
        out_shape=[
            jax.ShapeDtypeStruct((N, dout), jnp.float32),
            jax.ShapeDtypeStruct((N, 1), jnp.float32),
        ],
    )


def _tc_mid(din, dout):
    def body(raw_ref, y_ref, dinv_ref, b_ref, w_ref, out_ref):
        dinv = dinv_ref[...]
        acc = raw_ref[0] + raw_ref[1] + y_ref[...]
        X = jnp.maximum(acc * dinv + b_ref[...], 0.0)
        out_ref[...] = _dot(X, w_ref[...]) * dinv

    return pl.pallas_call(
        body,
        grid=(N // BR,),
        in_specs=[
            pl.BlockSpec((2, BR, din), lambda i: (0, i, 0)),
            pl.BlockSpec((BR, din), lambda i: (i, 0)),
            pl.BlockSpec((BR, 1), lambda i: (i, 0)),
            pl.BlockSpec((1, din), lambda i: (0, 0)),
            pl.BlockSpec((din, dout), lambda i: (0, 0)),
        ],
        out_specs=pl.BlockSpec((BR, dout), lambda i: (i, 0)),
        out_shape=jax.ShapeDtypeStruct((N, dout), jnp.float32),
    )


def _tc_last(din):
    def body(raw_ref, y_ref, dinv_ref, b_ref, out_ref):
        dinv = dinv_ref[...]
        acc = raw_ref[0] + raw_ref[1] + y_ref[...]
        X = jnp.maximum(acc * dinv + b_ref[...], 0.0)
        m = jnp.max(X, axis=1, keepdims=True)
        lse = jnp.log(jnp.sum(jnp.exp(X - m), axis=1, keepdims=True)) + m
        out_ref[...] = X - lse

    return pl.pallas_call(
        body,
        grid=(N // BR,),
        in_specs=[
            pl.BlockSpec((2, BR, din), lambda i: (0, i, 0)),
            pl.BlockSpec((BR, din), lambda i: (i, 0)),
            pl.BlockSpec((BR, 1), lambda i: (i, 0)),
            pl.BlockSpec((1, din), lambda i: (0, 0)),
        ],
        out_specs=pl.BlockSpec((BR, din), lambda i: (i, 0)),
        out_shape=jax.ShapeDtypeStruct((N, din), jnp.float32),
    )


def kernel(x, edge_index, W0, b0, W1, b1, W2, b2, W3, b3, W4, b4):
    src = edge_index[0]
    dst = edge_index[1]
    pad = E_PAD - E
    srcp = jnp.concatenate([src, jnp.zeros((pad,), jnp.int32)]).reshape(NW, NCHUNK, CH)
    dstp = jnp.concatenate([dst, jnp.full((pad,), N, jnp.int32)]).reshape(NW, NCHUNK, CH)
    onescol = jnp.concatenate(
        [jnp.ones((CH, 1), jnp.float32), jnp.zeros((CH, 15), jnp.float32)], axis=1)
    zeros = {d: jnp.zeros((AGG_ROWS, d), jnp.float32) for d in (128, 64, 32, 16)}

    degp = _deg_sc()(onescol, dstp, zeros[16])
    y, dinv = _tc_first(128, 128)(x, W0, degp)

    params = [(b0, W1, 64), (b1, W2, 32), (b2, W3, 16), (b3, W4, 16)]
    din = 128
    for b, W, dout in params:
        raw = _spmm_sc(din)(y, srcp, dstp, zeros[din])
        y = _tc_mid(din, dout)(raw, y, dinv, b.reshape(1, din), W)
        din = dout
    raw = _spmm_sc(din)(y, srcp, dstp, zeros[din])
    return _tc_last(din)(raw, y, dinv, b4.reshape(1, din))


# speedup vs baseline: 12.6168x; 1.0370x over previous
import functools

import jax
import jax.numpy as jnp
from jax import lax
from jax.experimental import pallas as pl
from jax.experimental.pallas import tpu as pltpu
from jax.experimental.pallas import tpu_sc as plsc

N = 10000
E = 320000
NC, NS = 2, 16
NW = NC * NS
CH = 128
NCHUNK = 79
E_PAD = NW * NCHUNK * CH
CH_N = 64
NCHUNK_N = 159
E_PAD_N = NW * NCHUNK_N * CH_N
AGG_ROWS = 10112
ZR = AGG_ROWS // NS
BR = 1000


def _spmm_sc(D):
    mesh = plsc.VectorSubcoreMesh(core_axis_name="c", subcore_axis_name="s")
    ch, nchunk = (CH_N, NCHUNK_N) if D >= 128 else (CH, NCHUNK)

    @functools.partial(
        pl.kernel,
        out_type=jax.ShapeDtypeStruct((NC, AGG_ROWS, D), jnp.float32),
        mesh=mesh,
        scratch_types=[
            pltpu.VMEM((nchunk, ch), jnp.int32),
            pltpu.VMEM((nchunk, ch), jnp.int32),
            pltpu.VMEM((ch, D), jnp.float32),
            pltpu.VMEM((ch, D), jnp.float32),
            pltpu.VMEM_SHARED((AGG_ROWS, D), jnp.float32),
            pltpu.SemaphoreType.DMA,
            pltpu.SemaphoreType.DMA,
        ],
        compiler_params=pltpu.CompilerParams(use_tc_tiling_on_sc=False),
    )
    def spmm(y_hbm, srcp_hbm, dstp_hbm, zeros_hbm, out_hbm,
             src_v, dst_v, buf0, buf1, agg, sem0, sem1):
        c = lax.axis_index("c")
        s = lax.axis_index("s")
        wid = s * NC + c
        pltpu.sync_copy(zeros_hbm.at[pl.ds(s * ZR, ZR)], agg.at[pl.ds(s * ZR, ZR)])
        pltpu.sync_copy(srcp_hbm.at[wid], src_v)
        pltpu.sync_copy(dstp_hbm.at[wid], dst_v)
        plsc.subcore_barrier()

        half = nchunk // 2
        pltpu.async_copy(y_hbm.at[src_v.at[0]], buf0, sem0)

        def body(j, carry):
            pltpu.async_copy(y_hbm.at[src_v.at[2 * j + 1]], buf1, sem1)
            pltpu.make_async_copy(y_hbm.at[src_v.at[2 * j]], buf0, sem0).wait()
            pltpu.sync_copy(buf0, agg.at[dst_v.at[2 * j]], add=True)
            pltpu.async_copy(y_hbm.at[src_v.at[2 * j + 2]], buf0, sem0)
            pltpu.make_async_copy(y_hbm.at[src_v.at[2 * j + 1]], buf1, sem1).wait()
            pltpu.sync_copy(buf1, agg.at[dst_v.at[2 * j + 1]], add=True)
            return carry

        lax.fori_loop(0, half, body, 0)
        pltpu.make_async_copy(y_hbm.at[src_v.at[nchunk - 1]], buf0, sem0).wait()
        pltpu.sync_copy(buf0, agg.at[dst_v.at[nchunk - 1]], add=True)
        plsc.subcore_barrier()
        pltpu.sync_copy(agg.at[pl.ds(s * ZR, ZR)], out_hbm.at[c, pl.ds(s * ZR, ZR)])

    return spmm


def _deg_sc():
    mesh = plsc.VectorSubcoreMesh(core_axis_name="c", subcore_axis_name="s")
    D = 16

    @functools.partial(
        pl.kernel,
        out_type=jax.ShapeDtypeStruct((NC, AGG_ROWS, D), jnp.float32),
        mesh=mesh,
        scratch_types=[
            pltpu.VMEM((NCHUNK, CH), jnp.int32),
            pltpu.VMEM((CH, D), jnp.float32),
            pltpu.VMEM_SHARED((AGG_ROWS, D), jnp.float32),
        ],
        compiler_params=pltpu.CompilerParams(use_tc_tiling_on_sc=False),
    )
    def degk(onescol_hbm, dstp_hbm, zeros_hbm, out_hbm, dst_v, buf, agg):
        c = lax.axis_index("c")
        s = lax.axis_index("s")
        wid = s * NC + c
        pltpu.sync_copy(zeros_hbm.at[pl.ds(s * ZR, ZR)], agg.at[pl.ds(s * ZR, ZR)])
        pltpu.sync_copy(onescol_hbm, buf)
        pltpu.sync_copy(dstp_hbm.at[wid], dst_v)
        plsc.subcore_barrier()

        def body(j, carry):
            pltpu.sync_copy(buf, agg.at[dst_v.at[j]], add=True)
            return carry

        lax.fori_loop(0, NCHUNK, body, 0)
        plsc.subcore_barrier()
        pltpu.sync_copy(agg.at[pl.ds(s * ZR, ZR)], out_hbm.at[c, pl.ds(s * ZR, ZR)])

    return degk


def _dot(a, b):
    return lax.dot_general(a, b, (((1,), (0,)), ((), ())),
                           precision=lax.Precision.HIGHEST,
                           preferred_element_type=jnp.float32)


def _tc_first(din, dout):
    def body(x_ref, w_ref, degs_ref, y_ref, dinv_ref):
        deg = degs_ref[0, :, 0:1] + degs_ref[1, :, 0:1] + 1.0
        dinv = lax.rsqrt(deg)
        y_ref[...] = _dot(x_ref[...], w_ref[...]) * dinv
        dinv_ref[...] = dinv

    return pl.pallas_call(
        body,
        grid=(N // BR,),
        in_specs=[
            pl.BlockSpec((BR, din), lambda i: (i, 0)),
            pl.BlockSpec((din, dout), lambda i: (0, 0)),
            pl.BlockSpec((2, BR, 16), lambda i: (0, i, 0)),
        ],
        out_specs=[
            pl.BlockSpec((BR, dout), lambda i: (i, 0)),
            pl.BlockSpec((BR, 1), lambda i: (i, 0)),
        ],
        out_shape=[
            jax.ShapeDtypeStruct((N, dout), jnp.float32),
            jax.ShapeDtypeStruct((N, 1), jnp.float32),
        ],
    )


def _tc_mid(din, dout):
    def body(raw_ref, y_ref, dinv_ref, b_ref, w_ref, out_ref):
        dinv = dinv_ref[...]
        acc = raw_ref[0] + raw_ref[1] + y_ref[...]
        X = jnp.maximum(acc * dinv + b_ref[...], 0.0)
        out_ref[...] = _dot(X, w_ref[...]) * dinv

    return pl.pallas_call(
        body,
        grid=(N // BR,),
        in_specs=[
            pl.BlockSpec((2, BR, din), lambda i: (0, i, 0)),
            pl.BlockSpec((BR, din), lambda i: (i, 0)),
            pl.BlockSpec((BR, 1), lambda i: (i, 0)),
            pl.BlockSpec((1, din), lambda i: (0, 0)),
            pl.BlockSpec((din, dout), lambda i: (0, 0)),
        ],
        out_specs=pl.BlockSpec((BR, dout), lambda i: (i, 0)),
        out_shape=jax.ShapeDtypeStruct((N, dout), jnp.float32),
    )


def _tc_last(din):
    def body(raw_ref, y_ref, dinv_ref, b_ref, out_ref):
        dinv = dinv_ref[...]
        acc = raw_ref[0] + raw_ref[1] + y_ref[...]
        X = jnp.maximum(acc * dinv + b_ref[...], 0.0)
        m = jnp.max(X, axis=1, keepdims=True)
        lse = jnp.log(jnp.sum(jnp.exp(X - m), axis=1, keepdims=True)) + m
        out_ref[...] = X - lse

    return pl.pallas_call(
        body,
        grid=(N // BR,),
        in_specs=[
            pl.BlockSpec((2, BR, din), lambda i: (0, i, 0)),
            pl.BlockSpec((BR, din), lambda i: (i, 0)),
            pl.BlockSpec((BR, 1), lambda i: (i, 0)),
            pl.BlockSpec((1, din), lambda i: (0, 0)),
        ],
        out_specs=pl.BlockSpec((BR, din), lambda i: (i, 0)),
        out_shape=jax.ShapeDtypeStruct((N, din), jnp.float32),
    )


def kernel(x, edge_index, W0, b0, W1, b1, W2, b2, W3, b3, W4, b4):
    src = edge_index[0]
    dst = edge_index[1]

    def lay(a, fill, e_pad, nchunk, ch):
        padv = jnp.full((e_pad - E,), fill, jnp.int32)
        return jnp.concatenate([a, padv]).reshape(NW, nchunk, ch)

    srcp = lay(src, 0, E_PAD, NCHUNK, CH)
    dstp = lay(dst, N, E_PAD, NCHUNK, CH)
    srcp_n = lay(src, 0, E_PAD_N, NCHUNK_N, CH_N)
    dstp_n = lay(dst, N, E_PAD_N, NCHUNK_N, CH_N)
    onescol = jnp.concatenate(
        [jnp.ones((CH, 1), jnp.float32), jnp.zeros((CH, 15), jnp.float32)], axis=1)
    zeros = {d: jnp.zeros((AGG_ROWS, d), jnp.float32) for d in (128, 64, 32, 16)}

    degp = _deg_sc()(onescol, dstp, zeros[16])
    y, dinv = _tc_first(128, 128)(x, W0, degp)

    params = [(b0, W1, 64), (b1, W2, 32), (b2, W3, 16), (b3, W4, 16)]
    din = 128
    for b, W, dout in params:
        sp, dp = (srcp_n, dstp_n) if din >= 128 else (srcp, dstp)
        raw = _spmm_sc(din)(y, sp, dp, zeros[din])
        y = _tc_mid(din, dout)(raw, y, dinv, b.reshape(1, din), W)
        din = dout
    raw = _spmm_sc(din)(y, srcp, dstp, zeros[din])
    return _tc_last(din)(raw, y, dinv, b4.reshape(1, din))
